# baseline (speedup 1.0000x reference)
import functools

import jax
import jax.numpy as jnp
from jax import lax
from jax.experimental import pallas as pl
from jax.experimental.pallas import tpu as pltpu
from jax.experimental.pallas import tpu_sc as plsc

_LOCAL_PERMS = (
    (1, 3, 5, 0, 2, 6, 7, 4),
    (2, 0, 4, 5, 7, 6, 3, 1),
    (5, 7, 0, 1, 4, 2, 6, 3),
    (4, 5, 3, 2, 7, 6, 0, 1),
    (6, 5, 1, 3, 2, 7, 4, 0),
    (3, 1, 7, 2, 4, 0, 5, 6),
    (1, 7, 6, 0, 5, 3, 4, 2),
    (5, 0, 2, 7, 1, 3, 4, 6),
)

_B, _C, _T = 128, 64, 4000
_NREG = 8
_RSZ = 8
_NBLK = _B * _NREG
_NW = 32
_BLK_PW = _NBLK // _NW


def _make_sc_permute():
    mesh = plsc.VectorSubcoreMesh(core_axis_name="c", subcore_axis_name="s")

    @functools.partial(
        pl.kernel,
        mesh=mesh,
        out_type=jax.ShapeDtypeStruct((_B * _C, _T), jnp.float32),
        scratch_types=[
            pltpu.VMEM((_RSZ, _T), jnp.float32),
            pltpu.SemaphoreType.DMA,
        ],
    )
    def sc_permute(in_hbm, out_hbm, buf, sem):
        wid = lax.axis_index("s") * 2 + lax.axis_index("c")
        base_blk = wid * _BLK_PW
        for o in range(_BLK_PW // _NREG):
            for r in range(_NREG):
                g = base_blk + o * _NREG + r
                row0 = pl.multiple_of(g * _RSZ, _RSZ)
                pltpu.sync_copy(in_hbm.at[pl.ds(row0, _RSZ), :], buf)
                lp = _LOCAL_PERMS[r]
                copies = [
                    pltpu.async_copy(
                        buf.at[pl.ds(lp[j], 1), :],
                        out_hbm.at[pl.ds(row0 + j, 1), :],
                        sem,
                    )
                    for j in range(_RSZ)
                ]
                for c in copies:
                    c.wait()

    return sc_permute


def kernel(data_tensor, domain_labels, aux_labels):
    B, C, one, T = data_tensor.shape
    flat = data_tensor.reshape(B * C, T)
    out = _make_sc_permute()(flat)
    return out.reshape(B, C, one, T)


# speedup vs baseline: 1.0877x; 1.0877x over previous
import functools

import jax
import jax.numpy as jnp
from jax import lax
from jax.experimental import pallas as pl
from jax.experimental.pallas import tpu as pltpu
from jax.experimental.pallas import tpu_sc as plsc

_LOCAL_PERMS = (
    (1, 3, 5, 0, 2, 6, 7, 4),
    (2, 0, 4, 5, 7, 6, 3, 1),
    (5, 7, 0, 1, 4, 2, 6, 3),
    (4, 5, 3, 2, 7, 6, 0, 1),
    (6, 5, 1, 3, 2, 7, 4, 0),
    (3, 1, 7, 2, 4, 0, 5, 6),
    (1, 7, 6, 0, 5, 3, 4, 2),
    (5, 0, 2, 7, 1, 3, 4, 6),
)

_B, _C, _T = 128, 64, 4000
_NREG = 8
_RSZ = 8
_NBLK = _B * _NREG
_NW = 32
_BLK_PW = _NBLK // _NW
_NOUT = _BLK_PW // _NREG


def _make_sc_permute():
    mesh = plsc.VectorSubcoreMesh(core_axis_name="c", subcore_axis_name="s")

    @functools.partial(
        pl.kernel,
        mesh=mesh,
        out_type=jax.ShapeDtypeStruct((_B * _C, _T), jnp.float32),
        scratch_types=[
            pltpu.VMEM((_RSZ, _T), jnp.float32),
            pltpu.VMEM((_RSZ, _T), jnp.float32),
            pltpu.SemaphoreType.DMA,
            pltpu.SemaphoreType.DMA,
            pltpu.SemaphoreType.DMA,
            pltpu.SemaphoreType.DMA,
        ],
    )
    def sc_permute(in_hbm, out_hbm, buf0, buf1, rs0, rs1, ws0, ws1):
        wid = lax.axis_index("s") * 2 + lax.axis_index("c")
        base_blk = wid * _BLK_PW
        bufs = (buf0, buf1)
        rsems = (rs0, rs1)
        wsems = (ws0, ws1)

        def do_block(o, k, first_two_guard):
            p = k % 2
            buf, rsem, wsem = bufs[p], rsems[p], wsems[p]
            g = base_blk + o * _NREG + k
            row0 = pl.multiple_of(g * _RSZ, _RSZ)
            drain = pltpu.make_async_copy(
                buf, out_hbm.at[pl.ds(0, _RSZ), :], wsem
            )
            if first_two_guard:
                @pl.when(o > 0)
                def _():
                    drain.wait()
            else:
                drain.wait()
            lp = _LOCAL_PERMS[k % _NREG]
            reads = [
                pltpu.async_copy(
                    in_hbm.at[pl.ds(row0 + lp[j], 1), :],
                    buf.at[pl.ds(j, 1), :],
                    rsem,
                )
                for j in range(_RSZ)
            ]
            for c in reads:
                c.wait()
            pltpu.async_copy(buf, out_hbm.at[pl.ds(row0, _RSZ), :], wsem)

        @pl.loop(0, _NOUT)
        def _outer(o):
            for k in range(_NREG):
                do_block(o, k, first_two_guard=(k < 2))

        for p in range(2):
            pltpu.make_async_copy(
                bufs[p], out_hbm.at[pl.ds(0, _RSZ), :], wsems[p]
            ).wait()

    return sc_permute


def kernel(data_tensor, domain_labels, aux_labels):
    B, C, one, T = data_tensor.shape
    flat = data_tensor.reshape(B * C, T)
    out = _make_sc_permute()(flat)
    return out.reshape(B, C, one, T)
